# baseline (speedup 1.0000x reference)
import jax
import jax.numpy as jnp
from jax import lax
from jax.experimental import pallas as pl
from jax.experimental.pallas import tpu as pltpu

_MAX_LEN = 199
_DIM = 128
_NUM_MASKS = 32
_BB = 256
_LB = 32


def _cl_kernel(ind_ref, masks_ref, xt_ref, xp_ref, out_ref):
    ind = ind_ref[...]
    kiota = lax.broadcasted_iota(jnp.int32, (_NUM_MASKS, _DIM), 0)
    onehot_kd = (kiota == jnp.broadcast_to(ind, (_NUM_MASKS, _DIM))).astype(
        jnp.float32
    )
    table = lax.dot_general(
        onehot_kd,
        masks_ref[...],
        dimension_numbers=(((0,), (0,)), ((), ())),
        preferred_element_type=jnp.float32,
    )

    rows = _BB * _LB
    xt = xt_ref[...].reshape(rows, _DIM)
    m = jnp.max(xt, axis=-1, keepdims=True)
    eqf = (xt == m).astype(jnp.float32)
    r = lax.broadcasted_iota(jnp.int32, (_DIM, _DIM), 0)
    c = lax.broadcasted_iota(jnp.int32, (_DIM, _DIM), 1)
    tri = (r < c).astype(jnp.float32)
    pf = jnp.dot(eqf, tri, preferred_element_type=jnp.float32)
    onehot = jnp.where(pf == 0.0, eqf, 0.0)

    mask_rows = jnp.dot(
        onehot, table, preferred_element_type=jnp.float32
    )
    p = jnp.exp(xp_ref[...]).reshape(rows, _DIM) * mask_rows
    denom = jnp.dot(
        p, jnp.ones((_DIM, _DIM), jnp.float32), preferred_element_type=jnp.float32
    )
    out_ref[...] = (p / denom).reshape(_BB, _LB, _DIM)


@jax.jit
def kernel(x_true, x_pred, masks, ind_of_ind):
    batch, seq, dim = x_true.shape
    grid = (batch // _BB, pl.cdiv(seq, _LB))
    return pl.pallas_call(
        _cl_kernel,
        grid=grid,
        in_specs=[
            pl.BlockSpec((1, _DIM), lambda b, l: (0, 0)),
            pl.BlockSpec((_NUM_MASKS, _DIM), lambda b, l: (0, 0)),
            pl.BlockSpec((_BB, _LB, _DIM), lambda b, l: (b, l, 0)),
            pl.BlockSpec((_BB, _LB, _DIM), lambda b, l: (b, l, 0)),
        ],
        out_specs=pl.BlockSpec((_BB, _LB, _DIM), lambda b, l: (b, l, 0)),
        out_shape=jax.ShapeDtypeStruct(x_true.shape, jnp.float32),
        compiler_params=pltpu.CompilerParams(
            dimension_semantics=("parallel", "parallel"),
            vmem_limit_bytes=64 * 1024 * 1024,
        ),
    )(
        ind_of_ind.reshape(1, _DIM).astype(jnp.int32),
        masks,
        x_true,
        x_pred,
    )


# speedup vs baseline: 3.0298x; 3.0298x over previous
import jax
import jax.numpy as jnp
from jax import lax
from jax.experimental import pallas as pl
from jax.experimental.pallas import tpu as pltpu

_MAX_LEN = 199
_DIM = 128
_NUM_MASKS = 32
_LB = 8
_BB = 512


def _cl_kernel(ind_ref, masks_ref, xt_ref, xp_ref, out_ref):
    ind = ind_ref[...]
    kiota = lax.broadcasted_iota(jnp.int32, (_NUM_MASKS, _DIM), 0)
    onehot_kd = (kiota == jnp.broadcast_to(ind, (_NUM_MASKS, _DIM))).astype(
        jnp.float32
    )
    table = lax.dot_general(
        onehot_kd,
        masks_ref[...],
        dimension_numbers=(((0,), (0,)), ((), ())),
        preferred_element_type=jnp.float32,
    )

    rows = _LB * _BB
    xt = xt_ref[...].reshape(rows, _DIM)
    m = jnp.max(xt, axis=-1, keepdims=True)
    eqf = (xt == m).astype(jnp.float32)
    r = lax.broadcasted_iota(jnp.int32, (_DIM, _DIM), 0)
    c = lax.broadcasted_iota(jnp.int32, (_DIM, _DIM), 1)
    tri = (r < c).astype(jnp.float32)
    pf = jnp.dot(eqf, tri, preferred_element_type=jnp.float32)
    onehot = jnp.where(pf == 0.0, eqf, 0.0)

    mask_rows = jnp.dot(
        onehot, table, preferred_element_type=jnp.float32
    )
    p = jnp.exp(xp_ref[...]).reshape(rows, _DIM) * mask_rows
    denom = jnp.dot(
        p, jnp.ones((_DIM, _DIM), jnp.float32), preferred_element_type=jnp.float32
    )
    out_ref[...] = (p / denom).reshape(_LB, _BB, _DIM)


@jax.jit
def kernel(x_true, x_pred, masks, ind_of_ind):
    batch, seq, dim = x_true.shape
    xt = jnp.transpose(x_true, (1, 0, 2))
    xp = jnp.transpose(x_pred, (1, 0, 2))
    grid = (pl.cdiv(seq, _LB), batch // _BB)
    out = pl.pallas_call(
        _cl_kernel,
        grid=grid,
        in_specs=[
            pl.BlockSpec((1, _DIM), lambda l, b: (0, 0)),
            pl.BlockSpec((_NUM_MASKS, _DIM), lambda l, b: (0, 0)),
            pl.BlockSpec((_LB, _BB, _DIM), lambda l, b: (l, b, 0)),
            pl.BlockSpec((_LB, _BB, _DIM), lambda l, b: (l, b, 0)),
        ],
        out_specs=pl.BlockSpec((_LB, _BB, _DIM), lambda l, b: (l, b, 0)),
        out_shape=jax.ShapeDtypeStruct((seq, batch, dim), jnp.float32),
        compiler_params=pltpu.CompilerParams(
            dimension_semantics=("parallel", "parallel"),
        ),
    )(
        ind_of_ind.reshape(1, _DIM).astype(jnp.int32),
        masks,
        xt,
        xp,
    )
    return jnp.transpose(out, (1, 0, 2))


# speedup vs baseline: 3.0883x; 1.0193x over previous
import jax
import jax.numpy as jnp
from jax import lax
from jax.experimental import pallas as pl
from jax.experimental.pallas import tpu as pltpu

_MAX_LEN = 199
_DIM = 128
_NUM_MASKS = 32
_LB = 8
_BB = 512


def _cl_kernel(ind_ref, masks_ref, xt_ref, xp_ref, out_ref):
    ind = ind_ref[...]
    kiota = lax.broadcasted_iota(jnp.int32, (_NUM_MASKS, _DIM), 0)
    onehot_kd = (kiota == jnp.broadcast_to(ind, (_NUM_MASKS, _DIM))).astype(
        jnp.float32
    )
    table = lax.dot_general(
        onehot_kd,
        masks_ref[...],
        dimension_numbers=(((0,), (0,)), ((), ())),
        preferred_element_type=jnp.float32,
    )

    rows = _LB * _BB
    xt = xt_ref[...].reshape(rows, _DIM)
    m = jnp.max(xt, axis=-1, keepdims=True)
    eqf = (xt == m).astype(jnp.bfloat16)
    r = lax.broadcasted_iota(jnp.int32, (_DIM, _DIM), 0)
    c = lax.broadcasted_iota(jnp.int32, (_DIM, _DIM), 1)
    tri = (r < c).astype(jnp.bfloat16)
    pf = jnp.dot(eqf, tri, preferred_element_type=jnp.float32)
    onehot = jnp.where(pf == 0.0, eqf, jnp.bfloat16(0.0))

    mask_rows = jnp.dot(
        onehot, table.astype(jnp.bfloat16), preferred_element_type=jnp.float32
    )
    p = jnp.exp(xp_ref[...]).reshape(rows, _DIM) * mask_rows
    denom = jnp.dot(
        p, jnp.ones((_DIM, _DIM), jnp.float32), preferred_element_type=jnp.float32
    )
    out_ref[...] = (p / denom).reshape(_LB, _BB, _DIM)


@jax.jit
def kernel(x_true, x_pred, masks, ind_of_ind):
    batch, seq, dim = x_true.shape
    xt = jnp.transpose(x_true, (1, 0, 2))
    xp = jnp.transpose(x_pred, (1, 0, 2))
    grid = (pl.cdiv(seq, _LB), batch // _BB)
    out = pl.pallas_call(
        _cl_kernel,
        grid=grid,
        in_specs=[
            pl.BlockSpec((1, _DIM), lambda l, b: (0, 0)),
            pl.BlockSpec((_NUM_MASKS, _DIM), lambda l, b: (0, 0)),
            pl.BlockSpec((_LB, _BB, _DIM), lambda l, b: (l, b, 0)),
            pl.BlockSpec((_LB, _BB, _DIM), lambda l, b: (l, b, 0)),
        ],
        out_specs=pl.BlockSpec((_LB, _BB, _DIM), lambda l, b: (l, b, 0)),
        out_shape=jax.ShapeDtypeStruct((seq, batch, dim), jnp.float32),
        compiler_params=pltpu.CompilerParams(
            dimension_semantics=("parallel", "parallel"),
        ),
    )(
        ind_of_ind.reshape(1, _DIM).astype(jnp.int32),
        masks,
        xt,
        xp,
    )
    return jnp.transpose(out, (1, 0, 2))


# speedup vs baseline: 3.1193x; 1.0100x over previous
import jax
import jax.numpy as jnp
from jax import lax
from jax.experimental import pallas as pl
from jax.experimental.pallas import tpu as pltpu

_MAX_LEN = 199
_DIM = 128
_NUM_MASKS = 32
_LB = 4
_BB = 1024


def _cl_kernel(ind_ref, masks_ref, xt_ref, xp_ref, out_ref):
    ind = ind_ref[...]
    kiota = lax.broadcasted_iota(jnp.int32, (_NUM_MASKS, _DIM), 0)
    onehot_kd = (kiota == jnp.broadcast_to(ind, (_NUM_MASKS, _DIM))).astype(
        jnp.float32
    )
    table = lax.dot_general(
        onehot_kd,
        masks_ref[...],
        dimension_numbers=(((0,), (0,)), ((), ())),
        preferred_element_type=jnp.float32,
    )

    rows = _LB * _BB
    xt = xt_ref[...].reshape(rows, _DIM)
    m = jnp.max(xt, axis=-1, keepdims=True)
    eqf = (xt == m).astype(jnp.bfloat16)
    r = lax.broadcasted_iota(jnp.int32, (_DIM, _DIM), 0)
    c = lax.broadcasted_iota(jnp.int32, (_DIM, _DIM), 1)
    tri = (r < c).astype(jnp.bfloat16)
    pf = jnp.dot(eqf, tri, preferred_element_type=jnp.float32)
    onehot = jnp.where(pf == 0.0, eqf, jnp.bfloat16(0.0))

    mask_rows = jnp.dot(
        onehot, table.astype(jnp.bfloat16), preferred_element_type=jnp.float32
    )
    p = jnp.exp(xp_ref[...]).reshape(rows, _DIM) * mask_rows
    denom = jnp.dot(
        p, jnp.ones((_DIM, _DIM), jnp.float32), preferred_element_type=jnp.float32
    )
    out_ref[...] = (p / denom).reshape(_LB, _BB, _DIM)


@jax.jit
def kernel(x_true, x_pred, masks, ind_of_ind):
    batch, seq, dim = x_true.shape
    xt = jnp.transpose(x_true, (1, 0, 2))
    xp = jnp.transpose(x_pred, (1, 0, 2))
    grid = (pl.cdiv(seq, _LB), batch // _BB)
    out = pl.pallas_call(
        _cl_kernel,
        grid=grid,
        in_specs=[
            pl.BlockSpec((1, _DIM), lambda l, b: (0, 0)),
            pl.BlockSpec((_NUM_MASKS, _DIM), lambda l, b: (0, 0)),
            pl.BlockSpec((_LB, _BB, _DIM), lambda l, b: (l, b, 0)),
            pl.BlockSpec((_LB, _BB, _DIM), lambda l, b: (l, b, 0)),
        ],
        out_specs=pl.BlockSpec((_LB, _BB, _DIM), lambda l, b: (l, b, 0)),
        out_shape=jax.ShapeDtypeStruct((seq, batch, dim), jnp.float32),
        compiler_params=pltpu.CompilerParams(
            dimension_semantics=("parallel", "parallel"),
        ),
    )(
        ind_of_ind.reshape(1, _DIM).astype(jnp.int32),
        masks,
        xt,
        xp,
    )
    return jnp.transpose(out, (1, 0, 2))


# speedup vs baseline: 3.3081x; 1.0605x over previous
import jax
import jax.numpy as jnp
from jax import lax
from jax.experimental import pallas as pl
from jax.experimental.pallas import tpu as pltpu

_MAX_LEN = 199
_DIM = 128
_NUM_MASKS = 32
_LB = 8
_BB = 1024


def _cl_kernel(ind_ref, masks_ref, xt_ref, xp_ref, out_ref):
    ind = ind_ref[...]
    kiota = lax.broadcasted_iota(jnp.int32, (_NUM_MASKS, _DIM), 0)
    onehot_kd = (kiota == jnp.broadcast_to(ind, (_NUM_MASKS, _DIM))).astype(
        jnp.float32
    )
    table = lax.dot_general(
        onehot_kd,
        masks_ref[...],
        dimension_numbers=(((0,), (0,)), ((), ())),
        preferred_element_type=jnp.float32,
    )

    rows = _LB * _BB
    xt = xt_ref[...].reshape(rows, _DIM)
    m = jnp.max(xt, axis=-1, keepdims=True)
    eqf = (xt == m).astype(jnp.bfloat16)
    r = lax.broadcasted_iota(jnp.int32, (_DIM, _DIM), 0)
    c = lax.broadcasted_iota(jnp.int32, (_DIM, _DIM), 1)
    tri = (r < c).astype(jnp.bfloat16)
    pf = jnp.dot(eqf, tri, preferred_element_type=jnp.float32)
    onehot = jnp.where(pf == 0.0, eqf, jnp.bfloat16(0.0))

    mask_rows = jnp.dot(
        onehot, table.astype(jnp.bfloat16), preferred_element_type=jnp.float32
    )
    p = jnp.exp(xp_ref[...]).reshape(rows, _DIM) * mask_rows
    denom = jnp.dot(
        p, jnp.ones((_DIM, _DIM), jnp.float32), preferred_element_type=jnp.float32
    )
    out_ref[...] = (p / denom).reshape(_LB, _BB, _DIM)


@jax.jit
def kernel(x_true, x_pred, masks, ind_of_ind):
    batch, seq, dim = x_true.shape
    xt = jnp.transpose(x_true, (1, 0, 2))
    xp = jnp.transpose(x_pred, (1, 0, 2))
    grid = (pl.cdiv(seq, _LB), batch // _BB)
    out = pl.pallas_call(
        _cl_kernel,
        grid=grid,
        in_specs=[
            pl.BlockSpec((1, _DIM), lambda l, b: (0, 0)),
            pl.BlockSpec((_NUM_MASKS, _DIM), lambda l, b: (0, 0)),
            pl.BlockSpec((_LB, _BB, _DIM), lambda l, b: (l, b, 0)),
            pl.BlockSpec((_LB, _BB, _DIM), lambda l, b: (l, b, 0)),
        ],
        out_specs=pl.BlockSpec((_LB, _BB, _DIM), lambda l, b: (l, b, 0)),
        out_shape=jax.ShapeDtypeStruct((seq, batch, dim), jnp.float32),
        compiler_params=pltpu.CompilerParams(
            dimension_semantics=("parallel", "parallel"),
        ),
    )(
        ind_of_ind.reshape(1, _DIM).astype(jnp.int32),
        masks,
        xt,
        xp,
    )
    return jnp.transpose(out, (1, 0, 2))


# speedup vs baseline: 3.3517x; 1.0132x over previous
import jax
import jax.numpy as jnp
from jax import lax
from jax.experimental import pallas as pl
from jax.experimental.pallas import tpu as pltpu

_MAX_LEN = 199
_DIM = 128
_NUM_MASKS = 32
_LB = 12
_BB = 1024


def _cl_kernel(ind_ref, masks_ref, xt_ref, xp_ref, out_ref):
    ind = ind_ref[...]
    kiota = lax.broadcasted_iota(jnp.int32, (_NUM_MASKS, _DIM), 0)
    onehot_kd = (kiota == jnp.broadcast_to(ind, (_NUM_MASKS, _DIM))).astype(
        jnp.float32
    )
    table = lax.dot_general(
        onehot_kd,
        masks_ref[...],
        dimension_numbers=(((0,), (0,)), ((), ())),
        preferred_element_type=jnp.float32,
    )

    rows = _LB * _BB
    xt = xt_ref[...].reshape(rows, _DIM)
    m = jnp.max(xt, axis=-1, keepdims=True)
    eqf = (xt == m).astype(jnp.bfloat16)
    r = lax.broadcasted_iota(jnp.int32, (_DIM, _DIM), 0)
    c = lax.broadcasted_iota(jnp.int32, (_DIM, _DIM), 1)
    tri = (r < c).astype(jnp.bfloat16)
    pf = jnp.dot(eqf, tri, preferred_element_type=jnp.float32)
    onehot = jnp.where(pf == 0.0, eqf, jnp.bfloat16(0.0))

    mask_rows = jnp.dot(
        onehot, table.astype(jnp.bfloat16), preferred_element_type=jnp.float32
    )
    p = jnp.exp(xp_ref[...]).reshape(rows, _DIM) * mask_rows
    denom = jnp.dot(
        p, jnp.ones((_DIM, _DIM), jnp.float32), preferred_element_type=jnp.float32
    )
    out_ref[...] = (p / denom).reshape(_LB, _BB, _DIM)


@jax.jit
def kernel(x_true, x_pred, masks, ind_of_ind):
    batch, seq, dim = x_true.shape
    xt = jnp.transpose(x_true, (1, 0, 2))
    xp = jnp.transpose(x_pred, (1, 0, 2))
    grid = (pl.cdiv(seq, _LB), batch // _BB)
    out = pl.pallas_call(
        _cl_kernel,
        grid=grid,
        in_specs=[
            pl.BlockSpec((1, _DIM), lambda l, b: (0, 0)),
            pl.BlockSpec((_NUM_MASKS, _DIM), lambda l, b: (0, 0)),
            pl.BlockSpec((_LB, _BB, _DIM), lambda l, b: (l, b, 0)),
            pl.BlockSpec((_LB, _BB, _DIM), lambda l, b: (l, b, 0)),
        ],
        out_specs=pl.BlockSpec((_LB, _BB, _DIM), lambda l, b: (l, b, 0)),
        out_shape=jax.ShapeDtypeStruct((seq, batch, dim), jnp.float32),
        compiler_params=pltpu.CompilerParams(
            dimension_semantics=("parallel", "parallel"),
        ),
    )(
        ind_of_ind.reshape(1, _DIM).astype(jnp.int32),
        masks,
        xt,
        xp,
    )
    return jnp.transpose(out, (1, 0, 2))
